# Initial kernel scaffold; baseline (speedup 1.0000x reference)
import jax
import jax.numpy as jnp
from jax.experimental import pallas as pl


def kernel(input, V, E, degV, degE, weight, bias):
    raise NotImplementedError("write your pallas kernel here")


# speedup vs baseline: 3.8018x; 3.8018x over previous
import functools

import jax
import jax.numpy as jnp
from jax import lax
from jax.experimental import pallas as pl
from jax.experimental.pallas import tpu as pltpu
from jax.experimental.pallas import tpu_sc as plsc

N_NODES = 10000
N_EDGES = 20000
NNZ = 320000
D = 128
H = 64
NTILE = 16
IDX_ROWS = 2560
T = IDX_ROWS // NTILE
NNZ_PAD = IDX_ROWS * 128
E_PAD = 20096
V_PAD = 10240
E_DUMMY = N_EDGES
EB = E_PAD // NTILE

_f32 = jnp.float32
_i32 = jnp.int32


def _matmul_body(x_ref, w_ref, x0_ref, x1_ref):
    acc = jnp.dot(x_ref[...], w_ref[...], preferred_element_type=_f32)
    x0_ref[...] = acc[:, :H]
    x1_ref[...] = acc[:, H:]


def _matmul(x, w):
    blk = 1000
    return pl.pallas_call(
        _matmul_body,
        grid=(N_NODES // blk,),
        in_specs=[
            pl.BlockSpec((blk, D), lambda i: (i, 0)),
            pl.BlockSpec((D, D), lambda i: (0, 0)),
        ],
        out_specs=[
            pl.BlockSpec((blk, H), lambda i: (i, 0)),
            pl.BlockSpec((blk, H), lambda i: (i, 0)),
        ],
        out_shape=[
            jax.ShapeDtypeStruct((N_NODES, H), _f32),
            jax.ShapeDtypeStruct((N_NODES, H), _f32),
        ],
    )(x, w)


G = 16


def _load_idx_group(ve2d, v_idx, e_idx, s, g):
    pltpu.sync_copy(ve2d.at[pl.ds(s * T + g * G, G)], v_idx)
    def body(j, carry):
        for k in range(8):
            sl = pl.ds(16 * k, 16)
            p16 = v_idx[j, sl]
            e_idx[j, sl] = lax.shift_right_logical(p16, 14)
            v_idx[j, sl] = jnp.bitwise_and(p16, 16383)
        return carry
    lax.fori_loop(0, G, body, 0)


def _edge_phase_body(x0, x1, ve2d, sums0, sums1, counts_out,
                     sums_sh, v_idx, e_idx, rows, cl, sem):
    c = lax.axis_index("c")
    s = lax.axis_index("s")
    zero16 = jnp.zeros((16,), _f32)
    ones16 = jnp.ones((16,), _f32)

    def zrows(r, carry):
        for k in range(4):
            rows[r, pl.ds(16 * k, 16)] = zero16
        return carry
    lax.fori_loop(0, 128, zrows, 0)

    def zcl(r, carry):
        cl[pl.ds(r * 16, 16)] = zero16
        return carry
    lax.fori_loop(0, E_PAD // 16, zcl, 0)

    def zs(j, carry):
        pltpu.sync_copy(rows, sums_sh.at[pl.ds(s * EB + j * 128, 128)])
        return carry
    lax.fori_loop(0, 9, zs, 0)
    pltpu.sync_copy(rows.at[pl.ds(0, 104)],
                    sums_sh.at[pl.ds(s * EB + 1152, 104)])

    plsc.subcore_barrier()

    def run(xsrc, do_count):
        def grp(g, carry):
            _load_idx_group(ve2d, v_idx, e_idx, s, g)
            def body(j, carry2):
                pltpu.async_copy(xsrc.at[v_idx.at[j]], rows, sem).wait()
                pltpu.sync_copy(rows, sums_sh.at[e_idx.at[j]], add=True)
                if do_count:
                    for k in range(8):
                        e16 = e_idx[j, pl.ds(16 * k, 16)]
                        plsc.addupdate_scatter(cl, [e16], ones16)
                return carry2
            lax.fori_loop(0, G, body, 0)
            return carry
        lax.fori_loop(0, T // G, grp, 0)

    @pl.when(c == 0)
    def _():
        run(x0, True)

    @pl.when(c == 1)
    def _():
        run(x1, False)

    plsc.subcore_barrier()

    def wout(dst):
        def body(j, carry):
            sl = pl.ds(s * EB + j * 128, 128)
            pltpu.sync_copy(sums_sh.at[sl], dst.at[sl])
            return carry
        lax.fori_loop(0, 9, body, 0)
        sl = pl.ds(s * EB + 1152, 104)
        pltpu.sync_copy(sums_sh.at[sl], dst.at[sl])

    @pl.when(c == 0)
    def _():
        wout(sums0)
        pltpu.sync_copy(cl, counts_out.at[s])

    @pl.when(c == 1)
    def _():
        wout(sums1)


def _edge_phase(x0, x1, ve2d):
    mesh = plsc.VectorSubcoreMesh(core_axis_name="c", subcore_axis_name="s")
    f = pl.kernel(
        _edge_phase_body,
        out_type=[
            jax.ShapeDtypeStruct((E_PAD, H), _f32),
            jax.ShapeDtypeStruct((E_PAD, H), _f32),
            jax.ShapeDtypeStruct((NTILE, E_PAD), _f32),
        ],
        mesh=mesh,
        compiler_params=pltpu.CompilerParams(use_tc_tiling_on_sc=False, needs_layout_passes=False),
        scratch_types=[
            pltpu.VMEM_SHARED((E_PAD, H), _f32),
            pltpu.VMEM((G, 128), _i32),
            pltpu.VMEM((G, 128), _i32),
            pltpu.VMEM((128, H), _f32),
            pltpu.VMEM((E_PAD,), _f32),
            pltpu.SemaphoreType.DMA,
        ],
    )
    return f(x0, x1, ve2d)


def _node_phase_body(xe0, xe1, ve2d, xv0, xv1,
                     xv_sh, v_idx, e_idx, rows, sem):
    c = lax.axis_index("c")
    s = lax.axis_index("s")
    zero16 = jnp.zeros((16,), _f32)

    def zrows(r, carry):
        for k in range(4):
            rows[r, pl.ds(16 * k, 16)] = zero16
        return carry
    lax.fori_loop(0, 128, zrows, 0)

    def zs(j, carry):
        pltpu.sync_copy(rows, xv_sh.at[pl.ds(s * 640 + j * 128, 128)])
        return carry
    lax.fori_loop(0, 5, zs, 0)

    plsc.subcore_barrier()

    def run(xsrc):
        def grp(g, carry):
            _load_idx_group(ve2d, v_idx, e_idx, s, g)
            def body(j, carry2):
                pltpu.async_copy(xsrc.at[e_idx.at[j]], rows, sem).wait()
                pltpu.sync_copy(rows, xv_sh.at[v_idx.at[j]], add=True)
                return carry2
            lax.fori_loop(0, G, body, 0)
            return carry
        lax.fori_loop(0, T // G, grp, 0)

    @pl.when(c == 0)
    def _():
        run(xe0)

    @pl.when(c == 1)
    def _():
        run(xe1)

    plsc.subcore_barrier()

    def wout(dst):
        def body(j, carry):
            sl = pl.ds(s * 640 + j * 128, 128)
            pltpu.sync_copy(xv_sh.at[sl], dst.at[sl])
            return carry
        lax.fori_loop(0, 5, body, 0)

    @pl.when(c == 0)
    def _():
        wout(xv0)

    @pl.when(c == 1)
    def _():
        wout(xv1)


def _node_phase(xe0, xe1, ve2d):
    mesh = plsc.VectorSubcoreMesh(core_axis_name="c", subcore_axis_name="s")
    f = pl.kernel(
        _node_phase_body,
        out_type=[
            jax.ShapeDtypeStruct((V_PAD, H), _f32),
            jax.ShapeDtypeStruct((V_PAD, H), _f32),
        ],
        mesh=mesh,
        compiler_params=pltpu.CompilerParams(use_tc_tiling_on_sc=False, needs_layout_passes=False),
        scratch_types=[
            pltpu.VMEM_SHARED((V_PAD, H), _f32),
            pltpu.VMEM((G, 128), _i32),
            pltpu.VMEM((G, 128), _i32),
            pltpu.VMEM((128, H), _f32),
            pltpu.SemaphoreType.DMA,
        ],
    )
    return f(xe0, xe1, ve2d)


def _edge_scale_body(s0_ref, s1_ref, cnt_ref, deg_ref, xe0_ref, xe1_ref):
    parts = cnt_ref[...]
    total = lax.dot_general(parts, jnp.ones((NTILE, 1), _f32),
                            (((0,), (0,)), ((), ())),
                            preferred_element_type=_f32)
    scale = deg_ref[...] / jnp.maximum(total, 1.0)
    xe0_ref[...] = s0_ref[...] * scale
    xe1_ref[...] = s1_ref[...] * scale


def _edge_scale(s0, s1, cnt, deg):
    blk = 128
    g = E_PAD // blk
    return pl.pallas_call(
        _edge_scale_body,
        grid=(g,),
        in_specs=[
            pl.BlockSpec((blk, H), lambda i: (i, 0)),
            pl.BlockSpec((blk, H), lambda i: (i, 0)),
            pl.BlockSpec((NTILE, blk), lambda i: (0, i)),
            pl.BlockSpec((blk, 1), lambda i: (i, 0)),
        ],
        out_specs=[
            pl.BlockSpec((blk, H), lambda i: (i, 0)),
            pl.BlockSpec((blk, H), lambda i: (i, 0)),
        ],
        out_shape=[
            jax.ShapeDtypeStruct((E_PAD, H), _f32),
            jax.ShapeDtypeStruct((E_PAD, H), _f32),
        ],
    )(s0, s1, cnt, deg)


def _final_body(a_ref, b_ref, dv_ref, bias_ref, out_ref):
    dv = dv_ref[...]
    out_ref[...] = (jnp.concatenate([a_ref[...] * dv, b_ref[...] * dv], axis=1)
                    + bias_ref[...])


def _final(xv0, xv1, degv, bias2d):
    blk = 1000
    return pl.pallas_call(
        _final_body,
        grid=(N_NODES // blk,),
        in_specs=[
            pl.BlockSpec((blk, H), lambda i: (i, 0)),
            pl.BlockSpec((blk, H), lambda i: (i, 0)),
            pl.BlockSpec((blk, 1), lambda i: (i, 0)),
            pl.BlockSpec((1, D), lambda i: (0, 0)),
        ],
        out_specs=pl.BlockSpec((blk, D), lambda i: (i, 0)),
        out_shape=jax.ShapeDtypeStruct((N_NODES, D), _f32),
    )(xv0, xv1, degv, bias2d)


def kernel(input, V, E, degV, degE, weight, bias):
    x0, x1 = _matmul(input, weight)

    v_pad = jnp.pad(V.astype(_i32), (0, NNZ_PAD - NNZ))
    e_pad = jnp.pad(E.astype(_i32), (0, NNZ_PAD - NNZ),
                    constant_values=E_DUMMY)
    ve2d = (jnp.left_shift(e_pad, 14) | v_pad).reshape(IDX_ROWS, 128)

    sums0, sums1, counts = _edge_phase(x0, x1, ve2d)

    deg = jnp.zeros((E_PAD, 1), _f32).at[:N_EDGES].set(degE)
    xe0, xe1 = _edge_scale(sums0, sums1, counts, deg)

    xv0, xv1 = _node_phase(xe0, xe1, ve2d)

    return _final(xv0[:N_NODES], xv1[:N_NODES], degV, bias.reshape(1, D))


# speedup vs baseline: 4.2848x; 1.1270x over previous
import functools

import jax
import jax.numpy as jnp
from jax import lax
from jax.experimental import pallas as pl
from jax.experimental.pallas import tpu as pltpu
from jax.experimental.pallas import tpu_sc as plsc

N_NODES = 10000
N_EDGES = 20000
NNZ = 320000
D = 128
H = 64
NTILE = 16
IDX_ROWS = 2560
T = IDX_ROWS // NTILE
NNZ_PAD = IDX_ROWS * 128
E_PAD = 20096
V_PAD = 10240
E_DUMMY = N_EDGES
EB = E_PAD // NTILE

_f32 = jnp.float32
_i32 = jnp.int32


def _matmul_body(x_ref, w_ref, x0_ref, x1_ref):
    acc = jnp.dot(x_ref[...], w_ref[...], preferred_element_type=_f32)
    x0_ref[...] = acc[:, :H]
    x1_ref[...] = acc[:, H:]


def _matmul(x, w):
    blk = 1000
    return pl.pallas_call(
        _matmul_body,
        grid=(N_NODES // blk,),
        in_specs=[
            pl.BlockSpec((blk, D), lambda i: (i, 0)),
            pl.BlockSpec((D, D), lambda i: (0, 0)),
        ],
        out_specs=[
            pl.BlockSpec((blk, H), lambda i: (i, 0)),
            pl.BlockSpec((blk, H), lambda i: (i, 0)),
        ],
        out_shape=[
            jax.ShapeDtypeStruct((N_NODES, H), _f32),
            jax.ShapeDtypeStruct((N_NODES, H), _f32),
        ],
    )(x, w)


G = 16


def _load_idx_group(ve2d, v_idx, e_idx, s, g):
    pltpu.sync_copy(ve2d.at[pl.ds(s * T + g * G, G)], v_idx)
    def body(j, carry):
        for k in range(8):
            sl = pl.ds(16 * k, 16)
            p16 = v_idx[j, sl]
            e_idx[j, sl] = lax.shift_right_logical(p16, 14)
            v_idx[j, sl] = jnp.bitwise_and(p16, 16383)
        return carry
    lax.fori_loop(0, G, body, 0)


def _edge_phase_body(x0, x1, ve2d, sums0, sums1, counts_out,
                     sums_sh, v_idx, e_idx, rows, rows2, cl, sem):
    c = lax.axis_index("c")
    s = lax.axis_index("s")
    zero16 = jnp.zeros((16,), _f32)
    ones16 = jnp.ones((16,), _f32)

    def zrows(r, carry):
        for k in range(4):
            rows[r, pl.ds(16 * k, 16)] = zero16
        return carry
    lax.fori_loop(0, 128, zrows, 0)

    def zcl(r, carry):
        cl[pl.ds(r * 16, 16)] = zero16
        return carry
    lax.fori_loop(0, E_PAD // 16, zcl, 0)

    def zs(j, carry):
        pltpu.sync_copy(rows, sums_sh.at[pl.ds(s * EB + j * 128, 128)])
        return carry
    lax.fori_loop(0, 9, zs, 0)
    pltpu.sync_copy(rows.at[pl.ds(0, 104)],
                    sums_sh.at[pl.ds(s * EB + 1152, 104)])

    plsc.subcore_barrier()

    bufs = (rows, rows2)

    def run(xsrc, do_count):
        def grp(g, carry):
            _load_idx_group(ve2d, v_idx, e_idx, s, g)
            cps = [None] * G
            cps[0] = pltpu.async_copy(xsrc.at[v_idx.at[0]], bufs[0], sem)
            for j in range(G):
                cps[j].wait()
                if j + 1 < G:
                    cps[j + 1] = pltpu.async_copy(
                        xsrc.at[v_idx.at[j + 1]], bufs[(j + 1) & 1], sem)
                if do_count:
                    for k in range(8):
                        e16 = e_idx[j, pl.ds(16 * k, 16)]
                        plsc.addupdate_scatter(cl, [e16], ones16)
                pltpu.sync_copy(bufs[j & 1], sums_sh.at[e_idx.at[j]], add=True)
            return carry
        lax.fori_loop(0, T // G, grp, 0)

    @pl.when(c == 0)
    def _():
        run(x0, True)

    @pl.when(c == 1)
    def _():
        run(x1, False)

    plsc.subcore_barrier()

    def wout(dst):
        def body(j, carry):
            sl = pl.ds(s * EB + j * 128, 128)
            pltpu.sync_copy(sums_sh.at[sl], dst.at[sl])
            return carry
        lax.fori_loop(0, 9, body, 0)
        sl = pl.ds(s * EB + 1152, 104)
        pltpu.sync_copy(sums_sh.at[sl], dst.at[sl])

    @pl.when(c == 0)
    def _():
        wout(sums0)
        pltpu.sync_copy(cl, counts_out.at[s])

    @pl.when(c == 1)
    def _():
        wout(sums1)


def _edge_phase(x0, x1, ve2d):
    mesh = plsc.VectorSubcoreMesh(core_axis_name="c", subcore_axis_name="s")
    f = pl.kernel(
        _edge_phase_body,
        out_type=[
            jax.ShapeDtypeStruct((E_PAD, H), _f32),
            jax.ShapeDtypeStruct((E_PAD, H), _f32),
            jax.ShapeDtypeStruct((NTILE, E_PAD), _f32),
        ],
        mesh=mesh,
        compiler_params=pltpu.CompilerParams(use_tc_tiling_on_sc=False, needs_layout_passes=False),
        scratch_types=[
            pltpu.VMEM_SHARED((E_PAD, H), _f32),
            pltpu.VMEM((G, 128), _i32),
            pltpu.VMEM((G, 128), _i32),
            pltpu.VMEM((128, H), _f32),
            pltpu.VMEM((128, H), _f32),
            pltpu.VMEM((E_PAD,), _f32),
            pltpu.SemaphoreType.DMA,
        ],
    )
    return f(x0, x1, ve2d)


def _node_phase_body(xe0, xe1, ve2d, xv0, xv1,
                     xv_sh, v_idx, e_idx, rows, rows2, sem):
    c = lax.axis_index("c")
    s = lax.axis_index("s")
    zero16 = jnp.zeros((16,), _f32)

    def zrows(r, carry):
        for k in range(4):
            rows[r, pl.ds(16 * k, 16)] = zero16
        return carry
    lax.fori_loop(0, 128, zrows, 0)

    def zs(j, carry):
        pltpu.sync_copy(rows, xv_sh.at[pl.ds(s * 640 + j * 128, 128)])
        return carry
    lax.fori_loop(0, 5, zs, 0)

    plsc.subcore_barrier()

    bufs = (rows, rows2)

    def run(xsrc):
        def grp(g, carry):
            _load_idx_group(ve2d, v_idx, e_idx, s, g)
            cps = [None] * G
            cps[0] = pltpu.async_copy(xsrc.at[e_idx.at[0]], bufs[0], sem)
            for j in range(G):
                cps[j].wait()
                if j + 1 < G:
                    cps[j + 1] = pltpu.async_copy(
                        xsrc.at[e_idx.at[j + 1]], bufs[(j + 1) & 1], sem)
                pltpu.sync_copy(bufs[j & 1], xv_sh.at[v_idx.at[j]], add=True)
            return carry
        lax.fori_loop(0, T // G, grp, 0)

    @pl.when(c == 0)
    def _():
        run(xe0)

    @pl.when(c == 1)
    def _():
        run(xe1)

    plsc.subcore_barrier()

    def wout(dst):
        def body(j, carry):
            sl = pl.ds(s * 640 + j * 128, 128)
            pltpu.sync_copy(xv_sh.at[sl], dst.at[sl])
            return carry
        lax.fori_loop(0, 5, body, 0)

    @pl.when(c == 0)
    def _():
        wout(xv0)

    @pl.when(c == 1)
    def _():
        wout(xv1)


def _node_phase(xe0, xe1, ve2d):
    mesh = plsc.VectorSubcoreMesh(core_axis_name="c", subcore_axis_name="s")
    f = pl.kernel(
        _node_phase_body,
        out_type=[
            jax.ShapeDtypeStruct((V_PAD, H), _f32),
            jax.ShapeDtypeStruct((V_PAD, H), _f32),
        ],
        mesh=mesh,
        compiler_params=pltpu.CompilerParams(use_tc_tiling_on_sc=False, needs_layout_passes=False),
        scratch_types=[
            pltpu.VMEM_SHARED((V_PAD, H), _f32),
            pltpu.VMEM((G, 128), _i32),
            pltpu.VMEM((G, 128), _i32),
            pltpu.VMEM((128, H), _f32),
            pltpu.VMEM((128, H), _f32),
            pltpu.SemaphoreType.DMA,
        ],
    )
    return f(xe0, xe1, ve2d)


def _edge_scale_body(s0_ref, s1_ref, cnt_ref, deg_ref, xe0_ref, xe1_ref):
    parts = cnt_ref[...]
    total = lax.dot_general(parts, jnp.ones((NTILE, 1), _f32),
                            (((0,), (0,)), ((), ())),
                            preferred_element_type=_f32)
    scale = deg_ref[...] / jnp.maximum(total, 1.0)
    xe0_ref[...] = s0_ref[...] * scale
    xe1_ref[...] = s1_ref[...] * scale


def _edge_scale(s0, s1, cnt, deg):
    blk = 128
    g = E_PAD // blk
    return pl.pallas_call(
        _edge_scale_body,
        grid=(g,),
        in_specs=[
            pl.BlockSpec((blk, H), lambda i: (i, 0)),
            pl.BlockSpec((blk, H), lambda i: (i, 0)),
            pl.BlockSpec((NTILE, blk), lambda i: (0, i)),
            pl.BlockSpec((blk, 1), lambda i: (i, 0)),
        ],
        out_specs=[
            pl.BlockSpec((blk, H), lambda i: (i, 0)),
            pl.BlockSpec((blk, H), lambda i: (i, 0)),
        ],
        out_shape=[
            jax.ShapeDtypeStruct((E_PAD, H), _f32),
            jax.ShapeDtypeStruct((E_PAD, H), _f32),
        ],
    )(s0, s1, cnt, deg)


def _final_body(a_ref, b_ref, dv_ref, bias_ref, out_ref):
    dv = dv_ref[...]
    out_ref[...] = (jnp.concatenate([a_ref[...] * dv, b_ref[...] * dv], axis=1)
                    + bias_ref[...])


def _final(xv0, xv1, degv, bias2d):
    blk = 1000
    return pl.pallas_call(
        _final_body,
        grid=(N_NODES // blk,),
        in_specs=[
            pl.BlockSpec((blk, H), lambda i: (i, 0)),
            pl.BlockSpec((blk, H), lambda i: (i, 0)),
            pl.BlockSpec((blk, 1), lambda i: (i, 0)),
            pl.BlockSpec((1, D), lambda i: (0, 0)),
        ],
        out_specs=pl.BlockSpec((blk, D), lambda i: (i, 0)),
        out_shape=jax.ShapeDtypeStruct((N_NODES, D), _f32),
    )(xv0, xv1, degv, bias2d)


def kernel(input, V, E, degV, degE, weight, bias):
    x0, x1 = _matmul(input, weight)

    v_pad = jnp.pad(V.astype(_i32), (0, NNZ_PAD - NNZ))
    e_pad = jnp.pad(E.astype(_i32), (0, NNZ_PAD - NNZ),
                    constant_values=E_DUMMY)
    ve2d = (jnp.left_shift(e_pad, 14) | v_pad).reshape(IDX_ROWS, 128)

    sums0, sums1, counts = _edge_phase(x0, x1, ve2d)

    deg = jnp.zeros((E_PAD, 1), _f32).at[:N_EDGES].set(degE)
    xe0, xe1 = _edge_scale(sums0, sums1, counts, deg)

    xv0, xv1 = _node_phase(xe0, xe1, ve2d)

    return _final(xv0[:N_NODES], xv1[:N_NODES], degV, bias.reshape(1, D))


# speedup vs baseline: 4.9034x; 1.1444x over previous
import functools

import jax
import jax.numpy as jnp
from jax import lax
from jax.experimental import pallas as pl
from jax.experimental.pallas import tpu as pltpu
from jax.experimental.pallas import tpu_sc as plsc

N_NODES = 10000
N_EDGES = 20000
NNZ = 320000
D = 128
H = 64
NTILE = 16
IDX_ROWS = 2560
T = IDX_ROWS // NTILE
NNZ_PAD = IDX_ROWS * 128
E_PAD = 20096
V_PAD = 10240
E_DUMMY = N_EDGES
EB = E_PAD // NTILE

_f32 = jnp.float32
_i32 = jnp.int32


def _matmul_body(x_ref, w_ref, x0_ref, x1_ref):
    acc = jnp.dot(x_ref[...], w_ref[...], preferred_element_type=_f32)
    x0_ref[...] = acc[:, :H]
    x1_ref[...] = acc[:, H:]


def _matmul(x, w):
    blk = 1000
    return pl.pallas_call(
        _matmul_body,
        grid=(N_NODES // blk,),
        in_specs=[
            pl.BlockSpec((blk, D), lambda i: (i, 0)),
            pl.BlockSpec((D, D), lambda i: (0, 0)),
        ],
        out_specs=[
            pl.BlockSpec((blk, H), lambda i: (i, 0)),
            pl.BlockSpec((blk, H), lambda i: (i, 0)),
        ],
        out_shape=[
            jax.ShapeDtypeStruct((N_NODES, H), _f32),
            jax.ShapeDtypeStruct((N_NODES, H), _f32),
        ],
    )(x, w)


G = 16


def _load_idx_group(ve2d, v_idx, e_idx, s, g):
    pltpu.sync_copy(ve2d.at[pl.ds(s * T + g * G, G)], v_idx)
    def body(j, carry):
        for k in range(8):
            sl = pl.ds(16 * k, 16)
            p16 = v_idx[j, sl]
            e_idx[j, sl] = lax.shift_right_logical(p16, 14)
            v_idx[j, sl] = jnp.bitwise_and(p16, 16383)
        return carry
    lax.fori_loop(0, G, body, 0)


def _edge_phase_body(x0, x1, ve2d, sums0, sums1, counts_out,
                     sums_sh, v_idx, e_idx, rows, rows2, rows3, cl, sem, sem2):
    c = lax.axis_index("c")
    s = lax.axis_index("s")
    zero16 = jnp.zeros((16,), _f32)
    ones16 = jnp.ones((16,), _f32)

    def zrows(r, carry):
        for k in range(4):
            rows[r, pl.ds(16 * k, 16)] = zero16
        return carry
    lax.fori_loop(0, 128, zrows, 0)

    def zcl(r, carry):
        cl[pl.ds(r * 16, 16)] = zero16
        return carry
    lax.fori_loop(0, E_PAD // 16, zcl, 0)

    def zs(j, carry):
        pltpu.sync_copy(rows, sums_sh.at[pl.ds(s * EB + j * 128, 128)])
        return carry
    lax.fori_loop(0, 9, zs, 0)
    pltpu.sync_copy(rows.at[pl.ds(0, 104)],
                    sums_sh.at[pl.ds(s * EB + 1152, 104)])

    plsc.subcore_barrier()

    bufs = (rows, rows2, rows3)

    def run(xsrc, do_count):
        def grp(g, carry):
            _load_idx_group(ve2d, v_idx, e_idx, s, g)
            gw = [None] * G
            sc = [None] * G
            gw[0] = pltpu.async_copy(xsrc.at[v_idx.at[0]], bufs[0], sem)
            gw[1] = pltpu.async_copy(xsrc.at[v_idx.at[1]], bufs[1], sem)
            for j in range(G):
                gw[j].wait()
                if do_count:
                    for k in range(8):
                        e16 = e_idx[j, pl.ds(16 * k, 16)]
                        plsc.addupdate_scatter(cl, [e16], ones16)
                sc[j] = pltpu.async_copy(bufs[j % 3],
                                         sums_sh.at[e_idx.at[j]], sem2,
                                         add=True)
                if j + 2 < G:
                    if j >= 1:
                        sc[j - 1].wait()
                    gw[j + 2] = pltpu.async_copy(
                        xsrc.at[v_idx.at[j + 2]], bufs[(j + 2) % 3], sem)
            sc[G - 2].wait()
            sc[G - 1].wait()
            return carry
        lax.fori_loop(0, T // G, grp, 0)

    @pl.when(c == 0)
    def _():
        run(x0, True)

    @pl.when(c == 1)
    def _():
        run(x1, False)

    plsc.subcore_barrier()

    def wout(dst):
        def body(j, carry):
            sl = pl.ds(s * EB + j * 128, 128)
            pltpu.sync_copy(sums_sh.at[sl], dst.at[sl])
            return carry
        lax.fori_loop(0, 9, body, 0)
        sl = pl.ds(s * EB + 1152, 104)
        pltpu.sync_copy(sums_sh.at[sl], dst.at[sl])

    @pl.when(c == 0)
    def _():
        wout(sums0)
        pltpu.sync_copy(cl, counts_out.at[s])

    @pl.when(c == 1)
    def _():
        wout(sums1)


def _edge_phase(x0, x1, ve2d):
    mesh = plsc.VectorSubcoreMesh(core_axis_name="c", subcore_axis_name="s")
    f = pl.kernel(
        _edge_phase_body,
        out_type=[
            jax.ShapeDtypeStruct((E_PAD, H), _f32),
            jax.ShapeDtypeStruct((E_PAD, H), _f32),
            jax.ShapeDtypeStruct((NTILE, E_PAD), _f32),
        ],
        mesh=mesh,
        compiler_params=pltpu.CompilerParams(use_tc_tiling_on_sc=False, needs_layout_passes=False),
        scratch_types=[
            pltpu.VMEM_SHARED((E_PAD, H), _f32),
            pltpu.VMEM((G, 128), _i32),
            pltpu.VMEM((G, 128), _i32),
            pltpu.VMEM((128, H), _f32),
            pltpu.VMEM((128, H), _f32),
            pltpu.VMEM((128, H), _f32),
            pltpu.VMEM((E_PAD,), _f32),
            pltpu.SemaphoreType.DMA,
            pltpu.SemaphoreType.DMA,
        ],
    )
    return f(x0, x1, ve2d)


def _node_phase_body(xe0, xe1, ve2d, xv0, xv1,
                     xv_sh, v_idx, e_idx, rows, rows2, rows3, sem, sem2):
    c = lax.axis_index("c")
    s = lax.axis_index("s")
    zero16 = jnp.zeros((16,), _f32)

    def zrows(r, carry):
        for k in range(4):
            rows[r, pl.ds(16 * k, 16)] = zero16
        return carry
    lax.fori_loop(0, 128, zrows, 0)

    def zs(j, carry):
        pltpu.sync_copy(rows, xv_sh.at[pl.ds(s * 640 + j * 128, 128)])
        return carry
    lax.fori_loop(0, 5, zs, 0)

    plsc.subcore_barrier()

    bufs = (rows, rows2, rows3)

    def run(xsrc):
        def grp(g, carry):
            _load_idx_group(ve2d, v_idx, e_idx, s, g)
            gw = [None] * G
            sc = [None] * G
            gw[0] = pltpu.async_copy(xsrc.at[e_idx.at[0]], bufs[0], sem)
            gw[1] = pltpu.async_copy(xsrc.at[e_idx.at[1]], bufs[1], sem)
            for j in range(G):
                gw[j].wait()
                sc[j] = pltpu.async_copy(bufs[j % 3],
                                         xv_sh.at[v_idx.at[j]], sem2,
                                         add=True)
                if j + 2 < G:
                    if j >= 1:
                        sc[j - 1].wait()
                    gw[j + 2] = pltpu.async_copy(
                        xsrc.at[e_idx.at[j + 2]], bufs[(j + 2) % 3], sem)
            sc[G - 2].wait()
            sc[G - 1].wait()
            return carry
        lax.fori_loop(0, T // G, grp, 0)

    @pl.when(c == 0)
    def _():
        run(xe0)

    @pl.when(c == 1)
    def _():
        run(xe1)

    plsc.subcore_barrier()

    def wout(dst):
        def body(j, carry):
            sl = pl.ds(s * 640 + j * 128, 128)
            pltpu.sync_copy(xv_sh.at[sl], dst.at[sl])
            return carry
        lax.fori_loop(0, 5, body, 0)

    @pl.when(c == 0)
    def _():
        wout(xv0)

    @pl.when(c == 1)
    def _():
        wout(xv1)


def _node_phase(xe0, xe1, ve2d):
    mesh = plsc.VectorSubcoreMesh(core_axis_name="c", subcore_axis_name="s")
    f = pl.kernel(
        _node_phase_body,
        out_type=[
            jax.ShapeDtypeStruct((V_PAD, H), _f32),
            jax.ShapeDtypeStruct((V_PAD, H), _f32),
        ],
        mesh=mesh,
        compiler_params=pltpu.CompilerParams(use_tc_tiling_on_sc=False, needs_layout_passes=False),
        scratch_types=[
            pltpu.VMEM_SHARED((V_PAD, H), _f32),
            pltpu.VMEM((G, 128), _i32),
            pltpu.VMEM((G, 128), _i32),
            pltpu.VMEM((128, H), _f32),
            pltpu.VMEM((128, H), _f32),
            pltpu.VMEM((128, H), _f32),
            pltpu.SemaphoreType.DMA,
            pltpu.SemaphoreType.DMA,
        ],
    )
    return f(xe0, xe1, ve2d)


def _edge_scale_body(s0_ref, s1_ref, cnt_ref, deg_ref, xe0_ref, xe1_ref):
    parts = cnt_ref[...]
    total = lax.dot_general(parts, jnp.ones((NTILE, 1), _f32),
                            (((0,), (0,)), ((), ())),
                            preferred_element_type=_f32)
    scale = deg_ref[...] / jnp.maximum(total, 1.0)
    xe0_ref[...] = s0_ref[...] * scale
    xe1_ref[...] = s1_ref[...] * scale


def _edge_scale(s0, s1, cnt, deg):
    blk = 128
    g = E_PAD // blk
    return pl.pallas_call(
        _edge_scale_body,
        grid=(g,),
        in_specs=[
            pl.BlockSpec((blk, H), lambda i: (i, 0)),
            pl.BlockSpec((blk, H), lambda i: (i, 0)),
            pl.BlockSpec((NTILE, blk), lambda i: (0, i)),
            pl.BlockSpec((blk, 1), lambda i: (i, 0)),
        ],
        out_specs=[
            pl.BlockSpec((blk, H), lambda i: (i, 0)),
            pl.BlockSpec((blk, H), lambda i: (i, 0)),
        ],
        out_shape=[
            jax.ShapeDtypeStruct((E_PAD, H), _f32),
            jax.ShapeDtypeStruct((E_PAD, H), _f32),
        ],
    )(s0, s1, cnt, deg)


def _final_body(a_ref, b_ref, dv_ref, bias_ref, out_ref):
    dv = dv_ref[...]
    out_ref[...] = (jnp.concatenate([a_ref[...] * dv, b_ref[...] * dv], axis=1)
                    + bias_ref[...])


def _final(xv0, xv1, degv, bias2d):
    blk = 1000
    return pl.pallas_call(
        _final_body,
        grid=(N_NODES // blk,),
        in_specs=[
            pl.BlockSpec((blk, H), lambda i: (i, 0)),
            pl.BlockSpec((blk, H), lambda i: (i, 0)),
            pl.BlockSpec((blk, 1), lambda i: (i, 0)),
            pl.BlockSpec((1, D), lambda i: (0, 0)),
        ],
        out_specs=pl.BlockSpec((blk, D), lambda i: (i, 0)),
        out_shape=jax.ShapeDtypeStruct((N_NODES, D), _f32),
    )(xv0, xv1, degv, bias2d)


def kernel(input, V, E, degV, degE, weight, bias):
    x0, x1 = _matmul(input, weight)

    v_pad = jnp.pad(V.astype(_i32), (0, NNZ_PAD - NNZ))
    e_pad = jnp.pad(E.astype(_i32), (0, NNZ_PAD - NNZ),
                    constant_values=E_DUMMY)
    ve2d = (jnp.left_shift(e_pad, 14) | v_pad).reshape(IDX_ROWS, 128)

    sums0, sums1, counts = _edge_phase(x0, x1, ve2d)

    deg = jnp.zeros((E_PAD, 1), _f32).at[:N_EDGES].set(degE)
    xe0, xe1 = _edge_scale(sums0, sums1, counts, deg)

    xv0, xv1 = _node_phase(xe0, xe1, ve2d)

    return _final(xv0[:N_NODES], xv1[:N_NODES], degV, bias.reshape(1, D))
